# baseline (speedup 1.0000x reference)
import jax
import jax.numpy as jnp
from jax import lax
from jax.experimental import pallas as pl
from jax.experimental.pallas import tpu as pltpu

_UNROLL = 32


def _node_proj_kernel(nodesf_ref, wpr_ref, bpm_ref, out_ref):
    nodes_p = lax.dot_general(nodesf_ref[...], wpr_ref[...],
                              (((1,), (1,)), ((), ())),
                              preferred_element_type=jnp.float32)
    out_ref[...] = nodes_p + bpm_ref[...]


def _skg_main_kernel(x_ref, np_ref, wgblk_ref, bgblk_ref, adj_ref,
                     wbpblk_ref, bbp_ref, out_ref):
    U = x_ref.shape[0]
    L = wgblk_ref.shape[0]

    Hs = [lax.dot_general(np_ref[...], x_ref[u], (((1,), (1,)), ((), ())),
                          preferred_element_type=jnp.float32)
          for u in range(U)]

    for l in range(L):
        hs = [jnp.dot(Hs[u], wgblk_ref[l],
                      preferred_element_type=jnp.float32)
              for u in range(U)]
        hs = [jnp.dot(adj_ref[...], h, preferred_element_type=jnp.float32)
              for h in hs]
        hs = [h + bgblk_ref[l] for h in hs]
        Hs = [jnp.maximum(h, 0.01 * h) for h in hs]

    for u in range(U):
        out = lax.dot_general(wbpblk_ref[...], Hs[u],
                              (((1,), (1,)), ((), ())),
                              preferred_element_type=jnp.float32)
        out_ref[u] = out + bbp_ref[0, 0]


def kernel(x, nodes, adj_hat, bbp, wp_rep, bp_m, wg_blk, bg_blk, wbp_blk):
    B, S, F = x.shape
    N, M, F2 = nodes.shape
    L, BS, _ = wg_blk.shape
    Bc = wbp_blk.shape[0]
    C = B // Bc
    U = _UNROLL if C % _UNROLL == 0 else 1

    nodes_flat = nodes.reshape(N, M * F2)
    nodes_p = pl.pallas_call(
        _node_proj_kernel,
        out_shape=jax.ShapeDtypeStruct((N, F), jnp.float32),
    )(nodes_flat, wp_rep, bp_m)

    x_chunks = x.reshape(C, BS, F)

    out = pl.pallas_call(
        _skg_main_kernel,
        out_shape=jax.ShapeDtypeStruct((C, Bc, N), jnp.float32),
        grid=(C // U,),
        in_specs=[
            pl.BlockSpec((U, BS, F), lambda c: (c, 0, 0)),
            pl.BlockSpec((N, F), lambda c: (0, 0)),
            pl.BlockSpec((L, BS, BS), lambda c: (0, 0, 0)),
            pl.BlockSpec((L, 1, BS), lambda c: (0, 0, 0)),
            pl.BlockSpec((N, N), lambda c: (0, 0)),
            pl.BlockSpec((Bc, BS), lambda c: (0, 0)),
            pl.BlockSpec(memory_space=pltpu.MemorySpace.SMEM),
        ],
        out_specs=pl.BlockSpec((U, Bc, N), lambda c: (c, 0, 0)),
        compiler_params=pltpu.CompilerParams(
            dimension_semantics=("parallel",)),
    )(x_chunks, nodes_p, wg_blk, bg_blk, adj_hat, wbp_blk, bbp)
    return out.reshape(B, N)


# speedup vs baseline: 1.0084x; 1.0084x over previous
import jax
import jax.numpy as jnp
from jax import lax
from jax.experimental import pallas as pl
from jax.experimental.pallas import tpu as pltpu

_UNROLL = 16


def _node_proj_kernel(nodesf_ref, wpr_ref, bpm_ref, out_ref):
    nodes_p = lax.dot_general(nodesf_ref[...], wpr_ref[...],
                              (((1,), (1,)), ((), ())),
                              preferred_element_type=jnp.float32)
    out_ref[...] = nodes_p + bpm_ref[...]


def _skg_main_kernel(x_ref, np_ref, wgblk_ref, bgblk_ref, adj_ref,
                     wbpblk_ref, bbp_ref, out_ref):
    U = x_ref.shape[0]
    L = wgblk_ref.shape[0]

    Hs = [lax.dot_general(np_ref[...], x_ref[u], (((1,), (1,)), ((), ())),
                          preferred_element_type=jnp.float32)
          for u in range(U)]

    for l in range(L):
        hs = [jnp.dot(Hs[u], wgblk_ref[l],
                      preferred_element_type=jnp.float32)
              for u in range(U)]
        hs = [jnp.dot(adj_ref[...], h, preferred_element_type=jnp.float32)
              for h in hs]
        hs = [h + bgblk_ref[l] for h in hs]
        Hs = [jnp.maximum(h, 0.01 * h) for h in hs]

    for u in range(U):
        out = lax.dot_general(wbpblk_ref[...], Hs[u],
                              (((1,), (1,)), ((), ())),
                              preferred_element_type=jnp.float32)
        out_ref[u] = out + bbp_ref[0, 0]


def kernel(x, nodes, adj_hat, bbp, wp_rep, bp_m, wg_blk, bg_blk, wbp_blk):
    B, S, F = x.shape
    N, M, F2 = nodes.shape
    L, BS, _ = wg_blk.shape
    Bc = wbp_blk.shape[0]
    C = B // Bc
    U = _UNROLL if C % _UNROLL == 0 else 1

    nodes_flat = nodes.reshape(N, M * F2)
    nodes_p = pl.pallas_call(
        _node_proj_kernel,
        out_shape=jax.ShapeDtypeStruct((N, F), jnp.float32),
    )(nodes_flat, wp_rep, bp_m)

    x_chunks = x.reshape(C, BS, F)

    out = pl.pallas_call(
        _skg_main_kernel,
        out_shape=jax.ShapeDtypeStruct((C, Bc, N), jnp.float32),
        grid=(C // U,),
        in_specs=[
            pl.BlockSpec((U, BS, F), lambda c: (c, 0, 0)),
            pl.BlockSpec((N, F), lambda c: (0, 0)),
            pl.BlockSpec((L, BS, BS), lambda c: (0, 0, 0)),
            pl.BlockSpec((L, 1, BS), lambda c: (0, 0, 0)),
            pl.BlockSpec((N, N), lambda c: (0, 0)),
            pl.BlockSpec((Bc, BS), lambda c: (0, 0)),
            pl.BlockSpec(memory_space=pltpu.MemorySpace.SMEM),
        ],
        out_specs=pl.BlockSpec((U, Bc, N), lambda c: (c, 0, 0)),
        compiler_params=pltpu.CompilerParams(
            dimension_semantics=("parallel",),
            flags={"XLA_TPU_STORE_TO_LOAD_FORWARDING_WINDOW": 12288}),
    )(x_chunks, nodes_p, wg_blk, bg_blk, adj_hat, wbp_blk, bbp)
    return out.reshape(B, N)
